# baseline (speedup 1.0000x reference)
import functools

import jax
import jax.numpy as jnp
from jax import lax
from jax.experimental import pallas as pl
from jax.experimental.pallas import tpu as pltpu
from jax.experimental.pallas import tpu_sc as plsc

N_WAY = 64
K_SHOT = 5
Q_QUERY = 20
D_IN = 2048
D_OUT = 1024
NQ = N_WAY * Q_QUERY
TOPK = 10
NEG_INF = -1e30
BLK = 256
_CPB = 8
_NE = N_WAY // _CPB
_NB = NQ // BLK

_F32 = jnp.float32


def _tc1_body(x_ref, w_ref, b_ref, qsim_ref, qry_ref, ap_ref, sup_s):
    pid = pl.program_id(0)

    @pl.when(pid < _NE)
    def _encode():
        o = jnp.dot(x_ref[...], w_ref[...], preferred_element_type=_F32) + b_ref[...]
        base_q = pl.multiple_of(pid * (_CPB * Q_QUERY), _CPB * Q_QUERY)
        base_c = pl.multiple_of(pid * _CPB, _CPB)
        qry_ref[pl.ds(base_q, _CPB * Q_QUERY), :] = jnp.concatenate(
            [o[kk * 25 + K_SHOT : (kk + 1) * 25, :] for kk in range(_CPB)], axis=0
        )
        sup_s[pl.ds(base_c, _CPB), :] = jnp.concatenate(
            [
                jnp.concatenate(
                    [o[kk * 25 + s : kk * 25 + s + 1, :] for s in range(K_SHOT)],
                    axis=1,
                )
                for kk in range(_CPB)
            ],
            axis=0,
        )

    @pl.when(pid >= _NE)
    def _sim():
        j = pid - _NE
        qall = qry_ref[...]
        qnt = lax.dot_general(
            jnp.ones((1, D_OUT), _F32), qall * qall, (((1,), (1,)), ((), ())),
            preferred_element_type=_F32,
        )

        qblk = qry_ref[pl.ds(pl.multiple_of(j * BLK, BLK), BLK), :]
        qnb = jnp.sum(qblk * qblk, axis=1, keepdims=True)
        qq = lax.dot_general(
            qblk, qall, (((1,), (1,)), ((), ())), preferred_element_type=_F32
        )
        qsim_ref[...] = 2.0 * qq - qnb - qnt

        @pl.when(j == 0)
        def _adapted_proto():
            proto = sup_s[:, 0 * D_OUT : 1 * D_OUT]
            for s in range(1, K_SHOT):
                proto = proto + sup_s[:, s * D_OUT : (s + 1) * D_OUT]
            proto = proto * (1.0 / K_SHOT)
            pn = jnp.sum(proto * proto, axis=1)
            pq = lax.dot_general(
                proto, qall, (((1,), (1,)), ((), ())), preferred_element_type=_F32
            )
            ps_t = 2.0 * pq - pn[:, None] - qnt

            row_n = lax.broadcasted_iota(jnp.int32, (N_WAY, NQ), 0).astype(_F32)
            m_col = jnp.max(ps_t, axis=0, keepdims=True)
            label = jnp.min(
                jnp.where(ps_t == m_col, row_n, jnp.float32(N_WAY)), axis=0,
                keepdims=True,
            )
            assign = row_n == label

            logq = jnp.where(assign, ps_t, NEG_INF)
            mx = jnp.maximum(jnp.max(logq, axis=1, keepdims=True), 0.0)
            e = jnp.exp(logq - mx)
            e_self = jnp.exp(-mx)
            den = jnp.sum(e, axis=1, keepdims=True) + e_self
            wq = e / den
            ap_ref[...] = (
                lax.dot_general(
                    wq, qall, (((1,), (0,)), ((), ())), preferred_element_type=_F32
                )
                + (e_self / den) * proto
            )


def _tc1(x, W, b):
    xblk = 25 * _CPB
    return pl.pallas_call(
        _tc1_body,
        grid=(_NE + _NB,),
        in_specs=[
            pl.BlockSpec((xblk, D_IN), lambda i: (jnp.minimum(i, _NE - 1), 0)),
            pl.BlockSpec((D_IN, D_OUT), lambda i: (0, 0)),
            pl.BlockSpec((1, D_OUT), lambda i: (0, 0)),
        ],
        out_specs=[
            pl.BlockSpec((BLK, NQ), lambda i: (jnp.maximum(i - _NE, 0), 0)),
            pl.BlockSpec((NQ, D_OUT), lambda i: (0, 0)),
            pl.BlockSpec((N_WAY, D_OUT), lambda i: (0, 0)),
        ],
        out_shape=[
            jax.ShapeDtypeStruct((NQ, NQ), _F32),
            jax.ShapeDtypeStruct((NQ, D_OUT), _F32),
            jax.ShapeDtypeStruct((N_WAY, D_OUT), _F32),
        ],
        scratch_shapes=[
            pltpu.VMEM((N_WAY, K_SHOT * D_OUT), _F32),
        ],
    )(x, W, b.reshape(1, D_OUT))


def _sc_topk(qsim):
    info = plsc.get_sparse_core_info()
    nc, ns = info.num_cores, info.num_subcores
    nw = nc * ns
    rpw = NQ // nw
    mesh = plsc.VectorSubcoreMesh(core_axis_name="c", subcore_axis_name="s")

    @functools.partial(
        pl.kernel,
        mesh=mesh,
        compiler_params=pltpu.CompilerParams(needs_layout_passes=False),
        out_type=[
            jax.ShapeDtypeStruct((NQ, 16), _F32),
            jax.ShapeDtypeStruct((NQ, 16), _F32),
        ],
        scratch_types=[
            pltpu.VMEM((rpw, NQ), _F32),
            pltpu.VMEM((rpw, 16), _F32),
            pltpu.VMEM((rpw, 16), _F32),
        ],
    )
    def k(qsim_hbm, v_out, i_out, rows_v, vbuf, ibuf):
        wid = lax.axis_index("s") * nc + lax.axis_index("c")
        base = wid * rpw
        pltpu.sync_copy(qsim_hbm.at[pl.ds(base, rpw)], rows_v)
        lanef = lax.iota(jnp.int32, 16).astype(_F32)

        def row_body(r, _):
            v0 = rows_v[r, pl.ds(0, 16)]
            cv0, ci0 = plsc.sort_key_val(v0, lanef, descending=True)
            cv0, ci0 = jnp.asarray(cv0), jnp.asarray(ci0)

            def chunk_body(c, carry):
                cv, ci = carry
                v = rows_v[r, pl.ds(c * 16, 16)]
                vi = lanef + (c * 16).astype(_F32)
                vs, isrt = plsc.sort_key_val(v, vi, descending=True)
                bv = lax.rev(cv, (0,))
                bi = lax.rev(ci, (0,))
                take = (vs > bv) | ((vs == bv) & (isrt < bi))
                mv = jnp.where(take, vs, bv)
                mi = jnp.where(take, isrt, bi)
                mv2, mi2 = plsc.sort_key_val(mv, mi, descending=True)
                return (mv2, mi2)

            cv, ci = lax.fori_loop(1, NQ // 16, chunk_body, (cv0, ci0))
            vbuf[r, :] = cv
            ibuf[r, :] = ci
            return 0

        lax.fori_loop(0, rpw, row_body, 0)
        pltpu.sync_copy(vbuf, v_out.at[pl.ds(base, rpw)])
        pltpu.sync_copy(ibuf, i_out.at[pl.ds(base, rpw)])

    return k(qsim)


def _fmt_body(v_ref, i_ref, idx_ref, idxt_ref):
    v = v_ref[...]
    ii = i_ref[...]
    col16 = lax.broadcasted_iota(jnp.int32, (BLK, 16), 1).astype(_F32)
    idx_mat = jnp.full((BLK, 16), float(NQ), dtype=_F32)
    for t in range(TOPK):
        mv = jnp.max(v, axis=1, keepdims=True)
        sel = jnp.min(
            jnp.where(v == mv, ii, jnp.float32(NQ)), axis=1, keepdims=True
        )
        idx_mat = jnp.where(col16 == float(t), sel, idx_mat)
        v = jnp.where(ii == sel, NEG_INF, v)
    idx_ref[...] = idx_mat
    idxt_ref[...] = idx_mat.T


def _fmt(v16, i16):
    return pl.pallas_call(
        _fmt_body,
        grid=(NQ // BLK,),
        in_specs=[
            pl.BlockSpec((BLK, 16), lambda i: (i, 0)),
            pl.BlockSpec((BLK, 16), lambda i: (i, 0)),
        ],
        out_specs=[
            pl.BlockSpec((BLK, 16), lambda i: (i, 0)),
            pl.BlockSpec((16, BLK), lambda i: (0, i)),
        ],
        out_shape=[
            jax.ShapeDtypeStruct((NQ, 16), _F32),
            jax.ShapeDtypeStruct((16, NQ), _F32),
        ],
    )(v16, i16)


def _comb_body(qsim_ref, idx_ref, idxt_ref, qall_ref, ap_ref, s_ref, o_ref):
    pid = pl.program_id(0)
    qsim = qsim_ref[...]
    col = lax.broadcasted_iota(jnp.int32, (BLK, NQ), 1).astype(_F32)

    m_blk = jnp.zeros((BLK, NQ), dtype=jnp.bool_)
    mt_blk = jnp.zeros((BLK, NQ), dtype=jnp.bool_)
    row_glob = lax.broadcasted_iota(jnp.int32, (BLK, NQ), 0).astype(
        _F32
    ) + jnp.float32(BLK) * pid.astype(_F32)
    for t in range(TOPK):
        m_blk = m_blk | (col == idx_ref[:, t : t + 1])
        mt_blk = mt_blk | (idxt_ref[t : t + 1, :] == row_glob)
    mutual = m_blk & mt_blk

    q_log = jnp.where(mutual, qsim, NEG_INF)
    mq = jnp.max(q_log, axis=1, keepdims=True)
    e = jnp.exp(q_log - mq)
    q_w = e / jnp.sum(e, axis=1, keepdims=True)

    aq = lax.dot_general(
        q_w, qall_ref[...], (((1,), (0,)), ((), ())), preferred_element_type=_F32
    )

    ap = ap_ref[...]
    apn = jnp.sum(ap * ap, axis=1)
    aqn = jnp.sum(aq * aq, axis=1, keepdims=True)
    aqp = lax.dot_general(
        aq, ap, (((1,), (1,)), ((), ())), preferred_element_type=_F32
    )
    sim = 2.0 * aqp - aqn - apn[None, :]
    o_ref[...] = s_ref[0] * sim + s_ref[1]


def _combine(qsim, idx, idxt, qry, ap, scal):
    return pl.pallas_call(
        _comb_body,
        grid=(NQ // BLK,),
        in_specs=[
            pl.BlockSpec((BLK, NQ), lambda i: (i, 0)),
            pl.BlockSpec((BLK, 16), lambda i: (i, 0)),
            pl.BlockSpec((16, NQ), lambda i: (0, 0)),
            pl.BlockSpec((NQ, D_OUT), lambda i: (0, 0)),
            pl.BlockSpec((N_WAY, D_OUT), lambda i: (0, 0)),
            pl.BlockSpec(memory_space=pltpu.SMEM),
        ],
        out_specs=pl.BlockSpec((BLK, N_WAY), lambda i: (i, 0)),
        out_shape=jax.ShapeDtypeStruct((NQ, N_WAY), _F32),
    )(qsim, idx, idxt, qry, ap, scal)


def kernel(x, W, b, tao, n, k, q):
    residual = (
        (jnp.asarray(n) - N_WAY)
        + (jnp.asarray(k) - K_SHOT)
        + (jnp.asarray(q) - Q_QUERY)
    ).astype(x.dtype)
    scal = jnp.stack([tao.astype(_F32), residual.astype(_F32)])
    qsim, qry, ap = _tc1(x, W, b)
    v16, i16 = _sc_topk(qsim)
    idx, idxt = _fmt(v16, i16)
    return _combine(qsim, idx, idxt, qry, ap, scal)


# speedup vs baseline: 1.8996x; 1.8996x over previous
import jax
import jax.numpy as jnp
from jax import lax
from jax.experimental import pallas as pl
from jax.experimental.pallas import tpu as pltpu

N_WAY = 64
K_SHOT = 5
Q_QUERY = 20
D_IN = 2048
D_OUT = 1024
NQ = N_WAY * Q_QUERY
TOPK = 10
NEG_INF = -1e30
BLK = 256
_CPB = 8
_NE = N_WAY // _CPB
_NB = NQ // BLK

_F32 = jnp.float32


def _body(x_ref, w_ref, b_ref, s_ref, o_ref,
          qry_s, sup_s, qsim_s, idx_s, idxt_s, ap_s, qnt_s):
    pid = pl.program_id(0)

    @pl.when(pid < _NE)
    def _encode():
        o = jnp.dot(x_ref[...], w_ref[...], preferred_element_type=_F32) + b_ref[...]
        base_q = pl.multiple_of(pid * (_CPB * Q_QUERY), _CPB * Q_QUERY)
        base_c = pl.multiple_of(pid * _CPB, _CPB)
        qry_s[pl.ds(base_q, _CPB * Q_QUERY), :] = jnp.concatenate(
            [o[kk * 25 + K_SHOT : (kk + 1) * 25, :] for kk in range(_CPB)], axis=0
        )
        sup_s[pl.ds(base_c, _CPB), :] = jnp.concatenate(
            [
                jnp.concatenate(
                    [o[kk * 25 + s : kk * 25 + s + 1, :] for s in range(K_SHOT)],
                    axis=1,
                )
                for kk in range(_CPB)
            ],
            axis=0,
        )

    @pl.when((pid >= _NE) & (pid < _NE + _NB))
    def _knn():
        j = pid - _NE
        qall = qry_s[...]

        @pl.when(j == 0)
        def _qnorms():
            qnt_s[...] = lax.dot_general(
                jnp.ones((1, D_OUT), _F32), qall * qall, (((1,), (1,)), ((), ())),
                preferred_element_type=_F32,
            )

        qnt = qnt_s[...]

        qblk = qry_s[pl.ds(pl.multiple_of(j * BLK, BLK), BLK), :]
        qnb = jnp.sum(qblk * qblk, axis=1, keepdims=True)
        qq = lax.dot_general(
            qblk, qall, (((1,), (1,)), ((), ())), preferred_element_type=_F32
        )
        qsim = 2.0 * qq - qnb - qnt
        qsim_s[pl.ds(pl.multiple_of(j * BLK, BLK), BLK), :] = qsim

        col = lax.broadcasted_iota(jnp.int32, (BLK, NQ), 1).astype(_F32)
        col16 = lax.broadcasted_iota(jnp.int32, (BLK, 16), 1).astype(_F32)
        idx_mat = jnp.full((BLK, 16), float(NQ), dtype=_F32)
        work = qsim
        for t in range(TOPK):
            mt = jnp.max(work, axis=1, keepdims=True)
            sel = jnp.min(
                jnp.where(work == mt, col, jnp.float32(NQ)), axis=1, keepdims=True
            )
            idx_mat = jnp.where(col16 == float(t), sel, idx_mat)
            work = jnp.where(col == sel, NEG_INF, work)
        idx_s[pl.ds(pl.multiple_of(j * BLK, BLK), BLK), :] = idx_mat
        idxt_s[:, pl.ds(pl.multiple_of(j * BLK, BLK), BLK)] = idx_mat.T

        @pl.when(j == 0)
        def _adapted_proto():
            proto = sup_s[:, 0 * D_OUT : 1 * D_OUT]
            for s in range(1, K_SHOT):
                proto = proto + sup_s[:, s * D_OUT : (s + 1) * D_OUT]
            proto = proto * (1.0 / K_SHOT)
            pn = jnp.sum(proto * proto, axis=1)
            pq = lax.dot_general(
                proto, qall, (((1,), (1,)), ((), ())), preferred_element_type=_F32
            )
            ps_t = 2.0 * pq - pn[:, None] - qnt

            row_n = lax.broadcasted_iota(jnp.int32, (N_WAY, NQ), 0).astype(_F32)
            m_col = jnp.max(ps_t, axis=0, keepdims=True)
            label = jnp.min(
                jnp.where(ps_t == m_col, row_n, jnp.float32(N_WAY)), axis=0,
                keepdims=True,
            )
            assign = row_n == label

            logq = jnp.where(assign, ps_t, NEG_INF)
            mx = jnp.maximum(jnp.max(logq, axis=1, keepdims=True), 0.0)
            e = jnp.exp(logq - mx)
            e_self = jnp.exp(-mx)
            den = jnp.sum(e, axis=1, keepdims=True) + e_self
            wq = e / den
            ap_s[...] = (
                lax.dot_general(
                    wq, qall, (((1,), (0,)), ((), ())), preferred_element_type=_F32
                )
                + (e_self / den) * proto
            )

    @pl.when(pid >= _NE + _NB)
    def _combine():
        j = pid - (_NE + _NB)
        qsim = qsim_s[pl.ds(pl.multiple_of(j * BLK, BLK), BLK), :]
        idx_mat = idx_s[pl.ds(pl.multiple_of(j * BLK, BLK), BLK), :]
        col = lax.broadcasted_iota(jnp.int32, (BLK, NQ), 1).astype(_F32)

        m_blk = jnp.zeros((BLK, NQ), dtype=jnp.bool_)
        mt_blk = jnp.zeros((BLK, NQ), dtype=jnp.bool_)
        row_glob = lax.broadcasted_iota(jnp.int32, (BLK, NQ), 0).astype(
            _F32
        ) + jnp.float32(BLK) * j.astype(_F32)
        for t in range(TOPK):
            m_blk = m_blk | (col == idx_mat[:, t : t + 1])
            mt_blk = mt_blk | (idxt_s[t : t + 1, :] == row_glob)
        mutual = m_blk & mt_blk

        q_log = jnp.where(mutual, qsim, NEG_INF)
        mq = jnp.max(q_log, axis=1, keepdims=True)
        e = jnp.exp(q_log - mq)
        q_w = e / jnp.sum(e, axis=1, keepdims=True)

        aq = lax.dot_general(
            q_w, qry_s[...], (((1,), (0,)), ((), ())), preferred_element_type=_F32
        )

        ap = ap_s[...]
        apn = jnp.sum(ap * ap, axis=1)
        aqn = jnp.sum(aq * aq, axis=1, keepdims=True)
        aqp = lax.dot_general(
            aq, ap, (((1,), (1,)), ((), ())), preferred_element_type=_F32
        )
        sim = 2.0 * aqp - aqn - apn[None, :]
        o_ref[...] = s_ref[0] * sim + s_ref[1]


def kernel(x, W, b, tao, n, k, q):
    residual = (
        (jnp.asarray(n) - N_WAY)
        + (jnp.asarray(k) - K_SHOT)
        + (jnp.asarray(q) - Q_QUERY)
    ).astype(x.dtype)
    scal = jnp.stack([tao.astype(_F32), residual.astype(_F32)])
    xblk = 25 * _CPB
    return pl.pallas_call(
        _body,
        grid=(_NE + 2 * _NB,),
        in_specs=[
            pl.BlockSpec((xblk, D_IN), lambda i: (jnp.minimum(i, _NE - 1), 0)),
            pl.BlockSpec((D_IN, D_OUT), lambda i: (0, 0)),
            pl.BlockSpec((1, D_OUT), lambda i: (0, 0)),
            pl.BlockSpec(memory_space=pltpu.SMEM),
        ],
        out_specs=pl.BlockSpec(
            (BLK, N_WAY), lambda i: (jnp.maximum(i - (_NE + _NB), 0), 0)
        ),
        out_shape=jax.ShapeDtypeStruct((NQ, N_WAY), _F32),
        scratch_shapes=[
            pltpu.VMEM((NQ, D_OUT), _F32),
            pltpu.VMEM((N_WAY, K_SHOT * D_OUT), _F32),
            pltpu.VMEM((NQ, NQ), _F32),
            pltpu.VMEM((NQ, 16), _F32),
            pltpu.VMEM((16, NQ), _F32),
            pltpu.VMEM((N_WAY, D_OUT), _F32),
            pltpu.VMEM((1, NQ), _F32),
        ],
    )(x, W, b.reshape(1, D_OUT), scal)
